# baseline (speedup 1.0000x reference)
import functools

import jax
import jax.numpy as jnp
from jax import lax
from jax.experimental import pallas as pl
from jax.experimental.pallas import tpu as pltpu
from jax.experimental.pallas import tpu_sc as plsc

E = 8
D = 1024
FF = 512
T = 2048
BLOCK = 256
NBMAX = (2 * T) // BLOCK + E - 1
P = NBMAX * BLOCK
XW = D + 128
NC = 2
NS = 16
NW = NC * NS
TPW = T // NW
BE_LEN = 48


def _dispatch_body(logits_ref, d0_ref, d1_ref, w0_ref, w1_ref, be_ref):
    logits = logits_ref[...].astype(jnp.float32)
    sub = lax.broadcasted_iota(jnp.int32, (E, T), 0)
    probs = jax.nn.softmax(logits, axis=0)
    m1 = jnp.max(probs, axis=0, keepdims=True)
    i1 = jnp.min(jnp.where(probs == m1, sub, E), axis=0, keepdims=True)
    probs2 = jnp.where(sub == i1, -1.0, probs)
    m2 = jnp.max(probs2, axis=0, keepdims=True)
    i2 = jnp.min(jnp.where(probs2 == m2, sub, E), axis=0, keepdims=True)
    s = m1 + m2
    w0_ref[...] = (m1 / s).reshape(T)
    w1_ref[...] = (m2 / s).reshape(T)

    mmat = ((sub == i1) | (sub == i2)).astype(jnp.int32)
    counts = jnp.sum(mmat, axis=1, keepdims=True)
    nb = (counts + (BLOCK - 1)) >> 8

    cum = nb
    for sh in (1, 2, 4):
        cum = cum + jnp.concatenate(
            [jnp.zeros((sh, 1), jnp.int32), cum[: E - sh, :]], axis=0)
    starts = cum - nb
    row_starts = starts * BLOCK

    inc = mmat
    sh = 1
    while sh < T:
        inc = inc + jnp.concatenate(
            [jnp.zeros((E, sh), jnp.int32), inc[:, : T - sh]], axis=1)
        sh *= 2
    prior = inc - mmat
    slot = prior + row_starts
    d0_ref[...] = jnp.sum(jnp.where(sub == i1, slot, 0), axis=0).reshape(T)
    d1_ref[...] = jnp.sum(jnp.where(sub == i2, slot, 0), axis=0).reshape(T)

    sub8 = lax.broadcasted_iota(jnp.int32, (E, 1), 0)
    b_iota = lax.broadcasted_iota(jnp.int32, (1, BE_LEN), 1)
    cnt = jnp.zeros((1, BE_LEN), jnp.int32)
    for e in range(E):
        s_e = jnp.sum(jnp.where(sub8 == e, starts, 0))
        cnt = cnt + (s_e <= b_iota).astype(jnp.int32)
    nbt = jnp.sum(nb)
    be_ref[...] = jnp.where(b_iota == BE_LEN - 1, nbt, cnt - 1).reshape(BE_LEN)


def _dispatch(logits_t):
    return pl.pallas_call(
        _dispatch_body,
        out_shape=(
            jax.ShapeDtypeStruct((T,), jnp.int32),
            jax.ShapeDtypeStruct((T,), jnp.int32),
            jax.ShapeDtypeStruct((T,), jnp.float32),
            jax.ShapeDtypeStruct((T,), jnp.float32),
            jax.ShapeDtypeStruct((BE_LEN,), jnp.int32),
        ),
    )(logits_t)


def _scatter_body(d0_hbm, d1_hbm, w0_hbm, w1_hbm, ts_hbm, ws_hbm,
                  tok_v, d0_v, d1_v, w0_v, w1_v, sem):
    wid = lax.axis_index("c") * NS + lax.axis_index("s")
    base = wid * TPW
    loads = [
        pltpu.async_copy(d0_hbm.at[pl.ds(base, TPW)], d0_v, sem),
        pltpu.async_copy(d1_hbm.at[pl.ds(base, TPW)], d1_v, sem),
        pltpu.async_copy(w0_hbm.at[pl.ds(base, TPW)], w0_v, sem),
        pltpu.async_copy(w1_hbm.at[pl.ds(base, TPW)], w1_v, sem),
    ]
    for g in range(TPW // 16):
        tok_v[pl.ds(g * 16, 16)] = lax.iota(jnp.int32, 16) + (base + g * 16)
    for cp in loads:
        cp.wait()
    stores = [
        pltpu.async_copy(tok_v, ts_hbm.at[d0_v], sem),
        pltpu.async_copy(tok_v, ts_hbm.at[d1_v], sem),
        pltpu.async_copy(w0_v, ws_hbm.at[d0_v], sem),
        pltpu.async_copy(w1_v, ws_hbm.at[d1_v], sem),
    ]
    for cp in stores:
        cp.wait()


def _scatter(d0, d1, w0, w1):
    mesh = plsc.VectorSubcoreMesh(
        core_axis_name="c", subcore_axis_name="s", num_cores=NC, num_subcores=NS)
    return pl.kernel(
        _scatter_body,
        out_type=(
            jax.ShapeDtypeStruct((P,), jnp.int32),
            jax.ShapeDtypeStruct((P,), jnp.float32),
        ),
        mesh=mesh,
        scratch_types=[
            pltpu.VMEM((TPW,), jnp.int32),
            pltpu.VMEM((TPW,), jnp.int32),
            pltpu.VMEM((TPW,), jnp.int32),
            pltpu.VMEM((TPW,), jnp.float32),
            pltpu.VMEM((TPW,), jnp.float32),
            pltpu.SemaphoreType.DMA,
        ],
    )(d0, d1, w0, w1)


def _gemm_body(be_ref, ts_ref, x_any, ws_ref, wu_ref, wg_ref, wd_ref, out_ref,
               xbuf0, xbuf1, sem0, sem1):
    b = pl.program_id(0)
    nbt = be_ref[BE_LEN - 1]

    def issue(blk, buf, sem):
        def row(i, _):
            t = ts_ref[blk * BLOCK + i]
            t = jnp.minimum(jnp.maximum(t, 0), T - 1)
            pltpu.make_async_copy(
                x_any.at[pl.ds(t, 1)], buf.at[pl.ds(i, 1)], sem).start()
            return 0
        lax.fori_loop(0, BLOCK, row, 0, unroll=False)

    @pl.when(b == 0)
    def _():
        issue(0, xbuf0, sem0)

    def stage(cur, csem, nxt, nsem):
        pltpu.make_async_copy(x_any.at[pl.ds(0, BLOCK)], cur, csem).wait()

        @pl.when(b + 1 < NBMAX)
        def _():
            issue(b + 1, nxt, nsem)

        @pl.when(b < nbt)
        def _():
            w_row = ws_ref[0]
            subl = lax.broadcasted_iota(jnp.int32, (BLOCK, BLOCK), 0)
            lanes = lax.broadcasted_iota(jnp.int32, (BLOCK, BLOCK), 1)
            diag = jnp.where(subl == lanes,
                             jnp.broadcast_to(w_row, (BLOCK, BLOCK)), 0.0)
            w = jnp.sum(diag, axis=1, keepdims=True)
            xc = cur[...].astype(jnp.bfloat16)
            wu = wu_ref[0].astype(jnp.bfloat16)
            wg = wg_ref[0].astype(jnp.bfloat16)
            wd = wd_ref[0].astype(jnp.bfloat16)
            dn = (((1,), (1,)), ((), ()))
            up = lax.dot_general(xc, wu, dn, preferred_element_type=jnp.float32)
            gate = lax.dot_general(xc, wg, dn, preferred_element_type=jnp.float32)
            gw = gate * w
            h = gw * jax.nn.sigmoid(gw) * (up * w)
            out_ref[...] = lax.dot_general(
                h.astype(jnp.bfloat16), wd, dn,
                preferred_element_type=jnp.float32) * w

    parity = lax.rem(b, 2)

    @pl.when(parity == 0)
    def _():
        stage(xbuf0, sem0, xbuf1, sem1)

    @pl.when(parity == 1)
    def _():
        stage(xbuf1, sem1, xbuf0, sem0)


def _gemm(be, ts, x, ws, w_up, w_gate, w_down):
    def clamp(b, be_ref, ts_ref):
        return jnp.minimum(b, be_ref[BE_LEN - 1] - 1)

    grid_spec = pltpu.PrefetchScalarGridSpec(
        num_scalar_prefetch=2,
        grid=(NBMAX,),
        in_specs=[
            pl.BlockSpec(memory_space=pl.ANY),
            pl.BlockSpec((1, 1, BLOCK), lambda b, be, ts: (clamp(b, be, ts), 0, 0)),
            pl.BlockSpec((1, FF, D), lambda b, be, ts: (be[clamp(b, be, ts)], 0, 0)),
            pl.BlockSpec((1, FF, D), lambda b, be, ts: (be[clamp(b, be, ts)], 0, 0)),
            pl.BlockSpec((1, D, FF), lambda b, be, ts: (be[clamp(b, be, ts)], 0, 0)),
        ],
        out_specs=pl.BlockSpec((BLOCK, D), lambda b, be, ts: (clamp(b, be, ts), 0)),
        scratch_shapes=[
            pltpu.VMEM((BLOCK, D), jnp.float32),
            pltpu.VMEM((BLOCK, D), jnp.float32),
            pltpu.SemaphoreType.DMA,
            pltpu.SemaphoreType.DMA,
        ],
    )
    return pl.pallas_call(
        _gemm_body,
        grid_spec=grid_spec,
        out_shape=jax.ShapeDtypeStruct((P, D), jnp.float32),
    )(be, ts, x, ws.reshape(NBMAX, 1, BLOCK), w_up, w_gate, w_down)


QTR = TPW // 4


def _combine_body(contrib_hbm, d0_hbm, d1_hbm, out_hbm,
                  p0_v, p1_v, g0a, g1a, g0b, g1b, oa, ob, sems):
    wid = lax.axis_index("c") * NS + lax.axis_index("s")
    base = wid * TPW
    pltpu.sync_copy(d0_hbm.at[pl.ds(base, TPW)], p0_v)
    pltpu.sync_copy(d1_hbm.at[pl.ds(base, TPW)], p1_v)
    gsets = [(g0a, g1a, oa), (g0b, g1b, ob)]

    def issue(q, gset):
        g0, g1, _ = gset
        return (
            pltpu.async_copy(contrib_hbm.at[p0_v.at[pl.ds(q * QTR, QTR)]], g0, sems[0]),
            pltpu.async_copy(contrib_hbm.at[p1_v.at[pl.ds(q * QTR, QTR)]], g1, sems[1]),
        )

    pend = issue(0, gsets[0])
    outcp = [None, None]
    for q in range(4):
        s = q % 2
        g0, g1, ob_ = gsets[s]
        for cp in pend:
            cp.wait()
        if q + 1 < 4:
            pend = issue(q + 1, gsets[1 - s])
        if outcp[s] is not None:
            outcp[s].wait()

        def chunk(i, _):
            sl = pl.ds(i * 16, 16)
            for r in range(QTR):
                ob_[r, sl] = g0[r, sl] + g1[r, sl]
            return 0

        lax.fori_loop(0, D // 16, chunk, 0, unroll=False)
        outcp[s] = pltpu.async_copy(
            ob_, out_hbm.at[pl.ds(base + q * QTR, QTR)], sems[2 + s])
    for cp in outcp:
        cp.wait()


def _combine(contrib, d0, d1):
    mesh = plsc.VectorSubcoreMesh(
        core_axis_name="c", subcore_axis_name="s", num_cores=NC, num_subcores=NS)
    return pl.kernel(
        _combine_body,
        out_type=jax.ShapeDtypeStruct((T, D), jnp.float32),
        mesh=mesh,
        scratch_types=[
            pltpu.VMEM((TPW,), jnp.int32),
            pltpu.VMEM((TPW,), jnp.int32),
            pltpu.VMEM((QTR, D), jnp.float32),
            pltpu.VMEM((QTR, D), jnp.float32),
            pltpu.VMEM((QTR, D), jnp.float32),
            pltpu.VMEM((QTR, D), jnp.float32),
            pltpu.VMEM((QTR, D), jnp.float32),
            pltpu.VMEM((QTR, D), jnp.float32),
            [pltpu.SemaphoreType.DMA] * 4,
        ],
    )(contrib, d0, d1)


@jax.jit
def kernel(x, router_logits_up, router_logits_gate, router_logits_down, w_up, w_gate, w_down):
    del router_logits_gate, router_logits_down
    d0, d1, w0, w1, be = _dispatch(router_logits_up.T)
    ts, ws = _scatter(d0, d1, w0, w1)
    contrib = _gemm(be, ts, x, ws, w_up, w_gate, w_down)
    return _combine(contrib, d0, d1)


# speedup vs baseline: 1.2877x; 1.2877x over previous
import functools

import jax
import jax.numpy as jnp
from jax import lax
from jax.experimental import pallas as pl
from jax.experimental.pallas import tpu as pltpu
from jax.experimental.pallas import tpu_sc as plsc

E = 8
D = 1024
FF = 512
T = 2048
BLOCK = 256
NBMAX = (2 * T) // BLOCK + E - 1
P = NBMAX * BLOCK
XW = D + 128
NC = 2
NS = 16
NW = NC * NS
TPW = T // NW
BE_LEN = 48


def _dispatch_body(logits_ref, d0_ref, d1_ref, w0_ref, w1_ref, be_ref):
    logits = logits_ref[...].astype(jnp.float32)
    sub = lax.broadcasted_iota(jnp.int32, (E, T), 0)
    probs = jax.nn.softmax(logits, axis=0)
    m1 = jnp.max(probs, axis=0, keepdims=True)
    i1 = jnp.min(jnp.where(probs == m1, sub, E), axis=0, keepdims=True)
    probs2 = jnp.where(sub == i1, -1.0, probs)
    m2 = jnp.max(probs2, axis=0, keepdims=True)
    i2 = jnp.min(jnp.where(probs2 == m2, sub, E), axis=0, keepdims=True)
    s = m1 + m2
    w0_ref[...] = (m1 / s).reshape(T)
    w1_ref[...] = (m2 / s).reshape(T)

    mmat = ((sub == i1) | (sub == i2)).astype(jnp.int32)
    counts = jnp.sum(mmat, axis=1, keepdims=True)
    nb = (counts + (BLOCK - 1)) >> 8

    cum = nb
    for sh in (1, 2, 4):
        cum = cum + jnp.concatenate(
            [jnp.zeros((sh, 1), jnp.int32), cum[: E - sh, :]], axis=0)
    starts = cum - nb
    row_starts = starts * BLOCK

    inc = mmat
    sh = 1
    while sh < T:
        inc = inc + jnp.concatenate(
            [jnp.zeros((E, sh), jnp.int32), inc[:, : T - sh]], axis=1)
        sh *= 2
    prior = inc - mmat
    slot = prior + row_starts
    d0_ref[...] = jnp.sum(jnp.where(sub == i1, slot, 0), axis=0).reshape(T)
    d1_ref[...] = jnp.sum(jnp.where(sub == i2, slot, 0), axis=0).reshape(T)

    sub8 = lax.broadcasted_iota(jnp.int32, (E, 1), 0)
    b_iota = lax.broadcasted_iota(jnp.int32, (1, BE_LEN), 1)
    cnt = jnp.zeros((1, BE_LEN), jnp.int32)
    for e in range(E):
        s_e = jnp.sum(jnp.where(sub8 == e, starts, 0))
        cnt = cnt + (s_e <= b_iota).astype(jnp.int32)
    nbt = jnp.sum(nb)
    be_ref[...] = jnp.where(b_iota == BE_LEN - 1, nbt, cnt - 1).reshape(BE_LEN)


def _dispatch(logits_t):
    return pl.pallas_call(
        _dispatch_body,
        out_shape=(
            jax.ShapeDtypeStruct((T,), jnp.int32),
            jax.ShapeDtypeStruct((T,), jnp.int32),
            jax.ShapeDtypeStruct((T,), jnp.float32),
            jax.ShapeDtypeStruct((T,), jnp.float32),
            jax.ShapeDtypeStruct((BE_LEN,), jnp.int32),
        ),
    )(logits_t)


def _scatter_body(x_hbm, d0_hbm, d1_hbm, w0_hbm, w1_hbm, xs_hbm, ws_hbm,
                  xrows, d0_v, d1_v, w0_v, w1_v, sem, wsem):
    wid = lax.axis_index("c") * NS + lax.axis_index("s")
    base = wid * TPW
    loads = [
        pltpu.async_copy(x_hbm.at[pl.ds(base, TPW)], xrows, sem),
        pltpu.async_copy(d0_hbm.at[pl.ds(base, TPW)], d0_v, wsem),
        pltpu.async_copy(d1_hbm.at[pl.ds(base, TPW)], d1_v, wsem),
        pltpu.async_copy(w0_hbm.at[pl.ds(base, TPW)], w0_v, wsem),
        pltpu.async_copy(w1_hbm.at[pl.ds(base, TPW)], w1_v, wsem),
    ]
    for cp in loads:
        cp.wait()

    stores = [
        pltpu.async_copy(w0_v, ws_hbm.at[d0_v], wsem),
        pltpu.async_copy(w1_v, ws_hbm.at[d1_v], wsem),
        pltpu.async_copy(xrows, xs_hbm.at[d0_v], sem),
        pltpu.async_copy(xrows, xs_hbm.at[d1_v], sem),
    ]
    for cp in stores:
        cp.wait()


def _scatter(xb, d0, d1, w0, w1):
    mesh = plsc.VectorSubcoreMesh(
        core_axis_name="c", subcore_axis_name="s", num_cores=NC, num_subcores=NS)
    return pl.kernel(
        _scatter_body,
        out_type=(
            jax.ShapeDtypeStruct((P, D), jnp.float32),
            jax.ShapeDtypeStruct((P,), jnp.float32),
        ),
        mesh=mesh,
        scratch_types=[
            pltpu.VMEM((TPW, D), jnp.float32),
            pltpu.VMEM((TPW,), jnp.int32),
            pltpu.VMEM((TPW,), jnp.int32),
            pltpu.VMEM((TPW,), jnp.float32),
            pltpu.VMEM((TPW,), jnp.float32),
            pltpu.SemaphoreType.DMA,
            pltpu.SemaphoreType.DMA,
        ],
    )(xb, d0, d1, w0, w1)


FSPLIT = 2


def _gemm_body(be_ref, xs_ref, ws_ref, wu_ref, wg_ref, wd_ref, out_ref):
    b = pl.program_id(0)
    s = pl.program_id(1)

    @pl.when(b < be_ref[BE_LEN - 1])
    def _():
        w_row = ws_ref[0]
        subl = lax.broadcasted_iota(jnp.int32, (BLOCK, BLOCK), 0)
        lanes = lax.broadcasted_iota(jnp.int32, (BLOCK, BLOCK), 1)
        diag = jnp.where(subl == lanes,
                         jnp.broadcast_to(w_row, (BLOCK, BLOCK)), 0.0)
        w = jnp.sum(diag, axis=1, keepdims=True)
        xc = xs_ref[...].astype(jnp.bfloat16)
        wu = wu_ref[0].astype(jnp.bfloat16)
        wg = wg_ref[0].astype(jnp.bfloat16)
        wd = wd_ref[0].astype(jnp.bfloat16)
        dn = (((1,), (1,)), ((), ()))
        up = lax.dot_general(xc, wu, dn, preferred_element_type=jnp.float32)
        gate = lax.dot_general(xc, wg, dn, preferred_element_type=jnp.float32)
        gw = gate * w
        h = gw * jax.nn.sigmoid(gw) * (up * w)
        part = lax.dot_general(
            h.astype(jnp.bfloat16), wd, dn,
            preferred_element_type=jnp.float32) * w

        @pl.when(s == 0)
        def _():
            out_ref[...] = part

        @pl.when(s != 0)
        def _():
            out_ref[...] += part


def _gemm(be, xs, ws, w_up, w_gate, w_down):
    def clamp(b, be_ref):
        return jnp.minimum(b, be_ref[BE_LEN - 1] - 1)

    fs = FF // FSPLIT
    grid_spec = pltpu.PrefetchScalarGridSpec(
        num_scalar_prefetch=1,
        grid=(NBMAX, FSPLIT),
        in_specs=[
            pl.BlockSpec((BLOCK, D), lambda b, s, be: (clamp(b, be), 0)),
            pl.BlockSpec((1, 1, BLOCK), lambda b, s, be: (clamp(b, be), 0, 0)),
            pl.BlockSpec((1, fs, D), lambda b, s, be: (be[clamp(b, be)], s, 0)),
            pl.BlockSpec((1, fs, D), lambda b, s, be: (be[clamp(b, be)], s, 0)),
            pl.BlockSpec((1, D, fs), lambda b, s, be: (be[clamp(b, be)], 0, s)),
        ],
        out_specs=pl.BlockSpec((BLOCK, D), lambda b, s, be: (clamp(b, be), 0)),
    )
    return pl.pallas_call(
        _gemm_body,
        grid_spec=grid_spec,
        out_shape=jax.ShapeDtypeStruct((P, D), jnp.float32),
    )(be, xs, ws.reshape(NBMAX, 1, BLOCK), w_up, w_gate, w_down)


QTR = TPW // 4


def _combine_body(contrib_hbm, d0_hbm, d1_hbm, out_hbm,
                  p0_v, p1_v, g0a, g1a, g0b, g1b, oa, ob, sems):
    wid = lax.axis_index("c") * NS + lax.axis_index("s")
    base = wid * TPW
    pltpu.sync_copy(d0_hbm.at[pl.ds(base, TPW)], p0_v)
    pltpu.sync_copy(d1_hbm.at[pl.ds(base, TPW)], p1_v)
    gsets = [(g0a, g1a, oa), (g0b, g1b, ob)]

    def issue(q, gset):
        g0, g1, _ = gset
        return (
            pltpu.async_copy(contrib_hbm.at[p0_v.at[pl.ds(q * QTR, QTR)]], g0, sems[0]),
            pltpu.async_copy(contrib_hbm.at[p1_v.at[pl.ds(q * QTR, QTR)]], g1, sems[1]),
        )

    pend = issue(0, gsets[0])
    outcp = [None, None]
    for q in range(4):
        s = q % 2
        g0, g1, ob_ = gsets[s]
        for cp in pend:
            cp.wait()
        if q + 1 < 4:
            pend = issue(q + 1, gsets[1 - s])
        if outcp[s] is not None:
            outcp[s].wait()

        def chunk(i, _):
            sl = pl.ds(i * 16, 16)
            for r in range(QTR):
                ob_[r, sl] = g0[r, sl] + g1[r, sl]
            return 0

        lax.fori_loop(0, D // 16, chunk, 0, unroll=False)
        outcp[s] = pltpu.async_copy(
            ob_, out_hbm.at[pl.ds(base + q * QTR, QTR)], sems[2 + s])
    for cp in outcp:
        cp.wait()


def _combine(contrib, d0, d1):
    mesh = plsc.VectorSubcoreMesh(
        core_axis_name="c", subcore_axis_name="s", num_cores=NC, num_subcores=NS)
    return pl.kernel(
        _combine_body,
        out_type=jax.ShapeDtypeStruct((T, D), jnp.float32),
        mesh=mesh,
        scratch_types=[
            pltpu.VMEM((TPW,), jnp.int32),
            pltpu.VMEM((TPW,), jnp.int32),
            pltpu.VMEM((QTR, D), jnp.float32),
            pltpu.VMEM((QTR, D), jnp.float32),
            pltpu.VMEM((QTR, D), jnp.float32),
            pltpu.VMEM((QTR, D), jnp.float32),
            pltpu.VMEM((QTR, D), jnp.float32),
            pltpu.VMEM((QTR, D), jnp.float32),
            [pltpu.SemaphoreType.DMA] * 4,
        ],
    )(contrib, d0, d1)


@jax.jit
def kernel(x, router_logits_up, router_logits_gate, router_logits_down, w_up, w_gate, w_down):
    del router_logits_gate, router_logits_down
    d0, d1, w0, w1, be = _dispatch(router_logits_up.T)
    xs, ws = _scatter(x, d0, d1, w0, w1)
    contrib = _gemm(be, xs, ws, w_up, w_gate, w_down)
    return _combine(contrib, d0, d1)


# speedup vs baseline: 1.5610x; 1.2122x over previous
import functools

import jax
import jax.numpy as jnp
from jax import lax
from jax.experimental import pallas as pl
from jax.experimental.pallas import tpu as pltpu
from jax.experimental.pallas import tpu_sc as plsc

E = 8
D = 1024
FF = 512
T = 2048
BLOCK = 256
NBMAX = (2 * T) // BLOCK + E - 1
P = NBMAX * BLOCK
XW = D + 128
NC = 2
NS = 16
NW = NC * NS
TPW = T // NW
BE_LEN = 48


def _dispatch_body(logits_ref, d0_ref, d1_ref, w0_ref, w1_ref, be_ref):
    logits = logits_ref[...].astype(jnp.float32)
    sub = lax.broadcasted_iota(jnp.int32, (E, T), 0)
    probs = jax.nn.softmax(logits, axis=0)
    m1 = jnp.max(probs, axis=0, keepdims=True)
    i1 = jnp.min(jnp.where(probs == m1, sub, E), axis=0, keepdims=True)
    probs2 = jnp.where(sub == i1, -1.0, probs)
    m2 = jnp.max(probs2, axis=0, keepdims=True)
    i2 = jnp.min(jnp.where(probs2 == m2, sub, E), axis=0, keepdims=True)
    s = m1 + m2
    w0_ref[...] = (m1 / s).reshape(T)
    w1_ref[...] = (m2 / s).reshape(T)

    mmat = ((sub == i1) | (sub == i2)).astype(jnp.int32)
    counts = jnp.sum(mmat, axis=1, keepdims=True)
    nb = (counts + (BLOCK - 1)) >> 8

    cum = nb
    for sh in (1, 2, 4):
        cum = cum + jnp.concatenate(
            [jnp.zeros((sh, 1), jnp.int32), cum[: E - sh, :]], axis=0)
    starts = cum - nb
    row_starts = starts * BLOCK

    inc = mmat
    sh = 1
    while sh < T:
        inc = inc + jnp.concatenate(
            [jnp.zeros((E, sh), jnp.int32), inc[:, : T - sh]], axis=1)
        sh *= 2
    prior = inc - mmat
    slot = prior + row_starts
    d0_ref[...] = jnp.sum(jnp.where(sub == i1, slot, 0), axis=0).reshape(T)
    d1_ref[...] = jnp.sum(jnp.where(sub == i2, slot, 0), axis=0).reshape(T)

    sub8 = lax.broadcasted_iota(jnp.int32, (E, 1), 0)
    b_iota = lax.broadcasted_iota(jnp.int32, (1, BE_LEN), 1)
    cnt = jnp.zeros((1, BE_LEN), jnp.int32)
    for e in range(E):
        s_e = jnp.sum(jnp.where(sub8 == e, starts, 0))
        cnt = cnt + (s_e <= b_iota).astype(jnp.int32)
    nbt = jnp.sum(nb)
    be_ref[...] = jnp.where(b_iota == BE_LEN - 1, nbt, cnt - 1).reshape(BE_LEN)


def _dispatch(logits_t):
    return pl.pallas_call(
        _dispatch_body,
        out_shape=(
            jax.ShapeDtypeStruct((T,), jnp.int32),
            jax.ShapeDtypeStruct((T,), jnp.int32),
            jax.ShapeDtypeStruct((T,), jnp.float32),
            jax.ShapeDtypeStruct((T,), jnp.float32),
            jax.ShapeDtypeStruct((BE_LEN,), jnp.int32),
        ),
    )(logits_t)


def _scatter_body(x_hbm, d0_hbm, d1_hbm, w0_hbm, w1_hbm, xs_hbm, ws_hbm,
                  xrows, d0_v, d1_v, w0_v, w1_v, sem, wsem):
    wid = lax.axis_index("c") * NS + lax.axis_index("s")
    base = wid * TPW
    loads = [
        pltpu.async_copy(x_hbm.at[pl.ds(base, TPW)], xrows, sem),
        pltpu.async_copy(d0_hbm.at[pl.ds(base, TPW)], d0_v, wsem),
        pltpu.async_copy(d1_hbm.at[pl.ds(base, TPW)], d1_v, wsem),
        pltpu.async_copy(w0_hbm.at[pl.ds(base, TPW)], w0_v, wsem),
        pltpu.async_copy(w1_hbm.at[pl.ds(base, TPW)], w1_v, wsem),
    ]
    for cp in loads:
        cp.wait()

    stores = [
        pltpu.async_copy(w0_v, ws_hbm.at[d0_v], wsem),
        pltpu.async_copy(w1_v, ws_hbm.at[d1_v], wsem),
        pltpu.async_copy(xrows, xs_hbm.at[d0_v], sem),
        pltpu.async_copy(xrows, xs_hbm.at[d1_v], sem),
    ]
    for cp in stores:
        cp.wait()


def _scatter(xb, d0, d1, w0, w1):
    mesh = plsc.VectorSubcoreMesh(
        core_axis_name="c", subcore_axis_name="s", num_cores=NC, num_subcores=NS)
    return pl.kernel(
        _scatter_body,
        out_type=(
            jax.ShapeDtypeStruct((P, D), jnp.float32),
            jax.ShapeDtypeStruct((P,), jnp.float32),
        ),
        mesh=mesh,
        scratch_types=[
            pltpu.VMEM((TPW, D), jnp.float32),
            pltpu.VMEM((TPW,), jnp.int32),
            pltpu.VMEM((TPW,), jnp.int32),
            pltpu.VMEM((TPW,), jnp.float32),
            pltpu.VMEM((TPW,), jnp.float32),
            pltpu.SemaphoreType.DMA,
            pltpu.SemaphoreType.DMA,
        ],
    )(xb, d0, d1, w0, w1)


FSPLIT = 1


def _gemm_body(be_ref, xs_ref, ws_ref, wu_ref, wg_ref, wd_ref, out_ref):
    b = pl.program_id(0)
    s = pl.program_id(1)

    @pl.when(b < be_ref[BE_LEN - 1])
    def _():
        w_row = ws_ref[0]
        subl = lax.broadcasted_iota(jnp.int32, (BLOCK, BLOCK), 0)
        lanes = lax.broadcasted_iota(jnp.int32, (BLOCK, BLOCK), 1)
        diag = jnp.where(subl == lanes,
                         jnp.broadcast_to(w_row, (BLOCK, BLOCK)), 0.0)
        w = jnp.sum(diag, axis=1, keepdims=True)
        xc = xs_ref[...].astype(jnp.bfloat16)
        wu = wu_ref[0].astype(jnp.bfloat16)
        wg = wg_ref[0].astype(jnp.bfloat16)
        wd = wd_ref[0].astype(jnp.bfloat16)
        dn = (((1,), (1,)), ((), ()))
        up = lax.dot_general(xc, wu, dn, preferred_element_type=jnp.float32)
        gate = lax.dot_general(xc, wg, dn, preferred_element_type=jnp.float32)
        gw = gate * w
        h = gw * jax.nn.sigmoid(gw) * (up * w)
        part = lax.dot_general(
            h.astype(jnp.bfloat16), wd, dn,
            preferred_element_type=jnp.float32) * w

        @pl.when(s == 0)
        def _():
            out_ref[...] = part

        @pl.when(s != 0)
        def _():
            out_ref[...] += part


def _gemm(be, xs, ws, w_up, w_gate, w_down):
    def clamp(b, be_ref):
        return jnp.minimum(b, be_ref[BE_LEN - 1] - 1)

    fs = FF // FSPLIT
    grid_spec = pltpu.PrefetchScalarGridSpec(
        num_scalar_prefetch=1,
        grid=(NBMAX, FSPLIT),
        in_specs=[
            pl.BlockSpec((BLOCK, D), lambda b, s, be: (clamp(b, be), 0)),
            pl.BlockSpec((1, 1, BLOCK), lambda b, s, be: (clamp(b, be), 0, 0)),
            pl.BlockSpec((1, fs, D), lambda b, s, be: (be[clamp(b, be)], s, 0)),
            pl.BlockSpec((1, fs, D), lambda b, s, be: (be[clamp(b, be)], s, 0)),
            pl.BlockSpec((1, D, fs), lambda b, s, be: (be[clamp(b, be)], 0, s)),
        ],
        out_specs=pl.BlockSpec((BLOCK, D), lambda b, s, be: (clamp(b, be), 0)),
    )
    return pl.pallas_call(
        _gemm_body,
        grid_spec=grid_spec,
        out_shape=jax.ShapeDtypeStruct((P, D), jnp.float32),
    )(be, xs, ws.reshape(NBMAX, 1, BLOCK), w_up, w_gate, w_down)


QTR = TPW // 4


def _combine_body(contrib_hbm, d0_hbm, d1_hbm, out_hbm,
                  p0_v, p1_v, g0a, g1a, g0b, g1b, oa, ob, sems):
    wid = lax.axis_index("c") * NS + lax.axis_index("s")
    base = wid * TPW
    pltpu.sync_copy(d0_hbm.at[pl.ds(base, TPW)], p0_v)
    pltpu.sync_copy(d1_hbm.at[pl.ds(base, TPW)], p1_v)
    gsets = [(g0a, g1a, oa), (g0b, g1b, ob)]

    def issue(q, gset):
        g0, g1, _ = gset
        return (
            pltpu.async_copy(contrib_hbm.at[p0_v.at[pl.ds(q * QTR, QTR)]], g0, sems[0]),
            pltpu.async_copy(contrib_hbm.at[p1_v.at[pl.ds(q * QTR, QTR)]], g1, sems[1]),
        )

    pend = issue(0, gsets[0])
    outcp = [None, None]
    for q in range(4):
        s = q % 2
        g0, g1, ob_ = gsets[s]
        for cp in pend:
            cp.wait()
        if q + 1 < 4:
            pend = issue(q + 1, gsets[1 - s])
        if outcp[s] is not None:
            outcp[s].wait()

        def chunk(i, _):
            sl = pl.ds(i * 16, 16)
            for r in range(QTR):
                ob_[r, sl] = g0[r, sl] + g1[r, sl]
            return 0

        lax.fori_loop(0, D // 16, chunk, 0, unroll=False)
        outcp[s] = pltpu.async_copy(
            ob_, out_hbm.at[pl.ds(base + q * QTR, QTR)], sems[2 + s])
    for cp in outcp:
        cp.wait()


def _combine(contrib, d0, d1):
    mesh = plsc.VectorSubcoreMesh(
        core_axis_name="c", subcore_axis_name="s", num_cores=NC, num_subcores=NS)
    return pl.kernel(
        _combine_body,
        out_type=jax.ShapeDtypeStruct((T, D), jnp.float32),
        mesh=mesh,
        scratch_types=[
            pltpu.VMEM((TPW,), jnp.int32),
            pltpu.VMEM((TPW,), jnp.int32),
            pltpu.VMEM((QTR, D), jnp.float32),
            pltpu.VMEM((QTR, D), jnp.float32),
            pltpu.VMEM((QTR, D), jnp.float32),
            pltpu.VMEM((QTR, D), jnp.float32),
            pltpu.VMEM((QTR, D), jnp.float32),
            pltpu.VMEM((QTR, D), jnp.float32),
            [pltpu.SemaphoreType.DMA] * 4,
        ],
    )(contrib, d0, d1)


@jax.jit
def kernel(x, router_logits_up, router_logits_gate, router_logits_down, w_up, w_gate, w_down):
    del router_logits_gate, router_logits_down
    d0, d1, w0, w1, be = _dispatch(router_logits_up.T)
    xs, ws = _scatter(x, d0, d1, w0, w1)
    contrib = _gemm(be, xs, ws, w_up, w_gate, w_down)
    return _combine(contrib, d0, d1)


# speedup vs baseline: 1.6387x; 1.0498x over previous
import functools

import jax
import jax.numpy as jnp
from jax import lax
from jax.experimental import pallas as pl
from jax.experimental.pallas import tpu as pltpu
from jax.experimental.pallas import tpu_sc as plsc

E = 8
D = 1024
FF = 512
T = 2048
BLOCK = 256
NBMAX = (2 * T) // BLOCK + E - 1
P = NBMAX * BLOCK
XW = D + 128
NC = 2
NS = 16
NW = NC * NS
TPW = T // NW
BE_LEN = 48


def _dispatch_body(logits_ref, d0_ref, d1_ref, w0_ref, w1_ref, be_ref):
    logits = logits_ref[...].astype(jnp.float32)
    sub = lax.broadcasted_iota(jnp.int32, (E, T), 0)
    probs = jax.nn.softmax(logits, axis=0)
    m1 = jnp.max(probs, axis=0, keepdims=True)
    i1 = jnp.min(jnp.where(probs == m1, sub, E), axis=0, keepdims=True)
    probs2 = jnp.where(sub == i1, -1.0, probs)
    m2 = jnp.max(probs2, axis=0, keepdims=True)
    i2 = jnp.min(jnp.where(probs2 == m2, sub, E), axis=0, keepdims=True)
    s = m1 + m2
    w0_ref[...] = (m1 / s).reshape(T)
    w1_ref[...] = (m2 / s).reshape(T)

    mmat = ((sub == i1) | (sub == i2)).astype(jnp.int32)
    counts = jnp.sum(mmat, axis=1, keepdims=True)
    nb = (counts + (BLOCK - 1)) >> 8

    cum = nb
    for sh in (1, 2, 4):
        cum = cum + jnp.concatenate(
            [jnp.zeros((sh, 1), jnp.int32), cum[: E - sh, :]], axis=0)
    starts = cum - nb
    row_starts = starts * BLOCK

    inc = mmat
    sh = 1
    while sh < T:
        inc = inc + jnp.concatenate(
            [jnp.zeros((E, sh), jnp.int32), inc[:, : T - sh]], axis=1)
        sh *= 2
    prior = inc - mmat
    slot = prior + row_starts
    d0_ref[...] = jnp.sum(jnp.where(sub == i1, slot, 0), axis=0).reshape(T)
    d1_ref[...] = jnp.sum(jnp.where(sub == i2, slot, 0), axis=0).reshape(T)

    sub8 = lax.broadcasted_iota(jnp.int32, (E, 1), 0)
    b_iota = lax.broadcasted_iota(jnp.int32, (1, BE_LEN), 1)
    cnt = jnp.zeros((1, BE_LEN), jnp.int32)
    for e in range(E):
        s_e = jnp.sum(jnp.where(sub8 == e, starts, 0))
        cnt = cnt + (s_e <= b_iota).astype(jnp.int32)
    nbt = jnp.sum(nb)
    be_ref[...] = jnp.where(b_iota == BE_LEN - 1, nbt, cnt - 1).reshape(BE_LEN)


def _dispatch(logits_t):
    return pl.pallas_call(
        _dispatch_body,
        out_shape=(
            jax.ShapeDtypeStruct((T,), jnp.int32),
            jax.ShapeDtypeStruct((T,), jnp.int32),
            jax.ShapeDtypeStruct((T,), jnp.float32),
            jax.ShapeDtypeStruct((T,), jnp.float32),
            jax.ShapeDtypeStruct((BE_LEN,), jnp.int32),
        ),
    )(logits_t)


HPW = TPW // 2


def _scatter_body(x_hbm, d0_hbm, d1_hbm, w0_hbm, w1_hbm, xs_hbm, ws_hbm,
                  xr0, xr1, d0a, d0b, d1a, d1b, w0a, w0b, w1a, w1b, sem, wsem):
    wid = lax.axis_index("c") * NS + lax.axis_index("s")
    base = wid * TPW
    lx0 = pltpu.async_copy(x_hbm.at[pl.ds(base, HPW)], xr0, sem)
    lx1 = pltpu.async_copy(x_hbm.at[pl.ds(base + HPW, HPW)], xr1, sem)
    smalls = [
        pltpu.async_copy(d0_hbm.at[pl.ds(base, HPW)], d0a, wsem),
        pltpu.async_copy(d0_hbm.at[pl.ds(base + HPW, HPW)], d0b, wsem),
        pltpu.async_copy(d1_hbm.at[pl.ds(base, HPW)], d1a, wsem),
        pltpu.async_copy(d1_hbm.at[pl.ds(base + HPW, HPW)], d1b, wsem),
        pltpu.async_copy(w0_hbm.at[pl.ds(base, HPW)], w0a, wsem),
        pltpu.async_copy(w0_hbm.at[pl.ds(base + HPW, HPW)], w0b, wsem),
        pltpu.async_copy(w1_hbm.at[pl.ds(base, HPW)], w1a, wsem),
        pltpu.async_copy(w1_hbm.at[pl.ds(base + HPW, HPW)], w1b, wsem),
    ]
    for cp in smalls:
        cp.wait()
    stores = [
        pltpu.async_copy(w0a, ws_hbm.at[d0a], wsem),
        pltpu.async_copy(w0b, ws_hbm.at[d0b], wsem),
        pltpu.async_copy(w1a, ws_hbm.at[d1a], wsem),
        pltpu.async_copy(w1b, ws_hbm.at[d1b], wsem),
    ]
    lx0.wait()
    stores += [
        pltpu.async_copy(xr0, xs_hbm.at[d0a], sem),
        pltpu.async_copy(xr0, xs_hbm.at[d1a], sem),
    ]
    lx1.wait()
    stores += [
        pltpu.async_copy(xr1, xs_hbm.at[d0b], sem),
        pltpu.async_copy(xr1, xs_hbm.at[d1b], sem),
    ]
    for cp in stores:
        cp.wait()


def _scatter(x, d0, d1, w0, w1):
    mesh = plsc.VectorSubcoreMesh(
        core_axis_name="c", subcore_axis_name="s", num_cores=NC, num_subcores=NS)
    return pl.kernel(
        _scatter_body,
        out_type=(
            jax.ShapeDtypeStruct((P, D), jnp.float32),
            jax.ShapeDtypeStruct((P,), jnp.float32),
        ),
        mesh=mesh,
        scratch_types=[
            pltpu.VMEM((HPW, D), jnp.float32),
            pltpu.VMEM((HPW, D), jnp.float32),
            pltpu.VMEM((HPW,), jnp.int32),
            pltpu.VMEM((HPW,), jnp.int32),
            pltpu.VMEM((HPW,), jnp.int32),
            pltpu.VMEM((HPW,), jnp.int32),
            pltpu.VMEM((HPW,), jnp.float32),
            pltpu.VMEM((HPW,), jnp.float32),
            pltpu.VMEM((HPW,), jnp.float32),
            pltpu.VMEM((HPW,), jnp.float32),
            pltpu.SemaphoreType.DMA,
            pltpu.SemaphoreType.DMA,
        ],
    )(x, d0, d1, w0, w1)


def _gemm_body(be_ref, xs_ref, ws_ref, wu_ref, wg_ref, wd_ref, out_ref):
    b = pl.program_id(0)

    @pl.when(b < be_ref[BE_LEN - 1])
    def _():
        w_row = ws_ref[0]
        subl = lax.broadcasted_iota(jnp.int32, (BLOCK, BLOCK), 0)
        lanes = lax.broadcasted_iota(jnp.int32, (BLOCK, BLOCK), 1)
        diag = jnp.where(subl == lanes,
                         jnp.broadcast_to(w_row, (BLOCK, BLOCK)), 0.0)
        w = jnp.sum(diag, axis=1, keepdims=True)
        xc = xs_ref[...].astype(jnp.bfloat16)
        wu = wu_ref[0].astype(jnp.bfloat16)
        wg = wg_ref[0].astype(jnp.bfloat16)
        wd = wd_ref[0].astype(jnp.bfloat16)
        dn = (((1,), (1,)), ((), ()))
        up = lax.dot_general(xc, wu, dn, preferred_element_type=jnp.float32)
        gate = lax.dot_general(xc, wg, dn, preferred_element_type=jnp.float32)
        gw = gate * w
        h = gw * jax.nn.sigmoid(gw) * (up * w)
        out_ref[...] = lax.dot_general(
            h.astype(jnp.bfloat16), wd, dn,
            preferred_element_type=jnp.float32) * w


def _gemm(be, xs, ws, w_up, w_gate, w_down):
    def clamp(b, be_ref):
        return jnp.minimum(b, be_ref[BE_LEN - 1] - 1)

    grid_spec = pltpu.PrefetchScalarGridSpec(
        num_scalar_prefetch=1,
        grid=(NBMAX,),
        in_specs=[
            pl.BlockSpec((BLOCK, D), lambda b, be: (clamp(b, be), 0)),
            pl.BlockSpec((1, 1, BLOCK), lambda b, be: (clamp(b, be), 0, 0)),
            pl.BlockSpec((1, FF, D), lambda b, be: (be[clamp(b, be)], 0, 0)),
            pl.BlockSpec((1, FF, D), lambda b, be: (be[clamp(b, be)], 0, 0)),
            pl.BlockSpec((1, D, FF), lambda b, be: (be[clamp(b, be)], 0, 0)),
        ],
        out_specs=pl.BlockSpec((BLOCK, D), lambda b, be: (clamp(b, be), 0)),
    )
    return pl.pallas_call(
        _gemm_body,
        grid_spec=grid_spec,
        out_shape=jax.ShapeDtypeStruct((P, D), jnp.float32),
    )(be, xs, ws.reshape(NBMAX, 1, BLOCK), w_up, w_gate, w_down)


QTR = TPW // 4


def _combine_body(contrib_hbm, d0_hbm, d1_hbm, out_hbm,
                  p0_v, p1_v, g0a, g1a, g0b, g1b, oa, ob, sems):
    wid = lax.axis_index("c") * NS + lax.axis_index("s")
    base = wid * TPW
    pltpu.sync_copy(d0_hbm.at[pl.ds(base, TPW)], p0_v)
    pltpu.sync_copy(d1_hbm.at[pl.ds(base, TPW)], p1_v)
    gsets = [(g0a, g1a, oa), (g0b, g1b, ob)]

    def issue(q, gset):
        g0, g1, _ = gset
        return (
            pltpu.async_copy(contrib_hbm.at[p0_v.at[pl.ds(q * QTR, QTR)]], g0, sems[0]),
            pltpu.async_copy(contrib_hbm.at[p1_v.at[pl.ds(q * QTR, QTR)]], g1, sems[1]),
        )

    pend = issue(0, gsets[0])
    outcp = [None, None]
    for q in range(4):
        s = q % 2
        g0, g1, ob_ = gsets[s]
        for cp in pend:
            cp.wait()
        if q + 1 < 4:
            pend = issue(q + 1, gsets[1 - s])
        if outcp[s] is not None:
            outcp[s].wait()

        def chunk(i, _):
            sl = pl.ds(i * 16, 16)
            for r in range(QTR):
                ob_[r, sl] = g0[r, sl] + g1[r, sl]
            return 0

        lax.fori_loop(0, D // 16, chunk, 0, unroll=False)
        outcp[s] = pltpu.async_copy(
            ob_, out_hbm.at[pl.ds(base + q * QTR, QTR)], sems[2 + s])
    for cp in outcp:
        cp.wait()


def _combine(contrib, d0, d1):
    mesh = plsc.VectorSubcoreMesh(
        core_axis_name="c", subcore_axis_name="s", num_cores=NC, num_subcores=NS)
    return pl.kernel(
        _combine_body,
        out_type=jax.ShapeDtypeStruct((T, D), jnp.float32),
        mesh=mesh,
        scratch_types=[
            pltpu.VMEM((TPW,), jnp.int32),
            pltpu.VMEM((TPW,), jnp.int32),
            pltpu.VMEM((QTR, D), jnp.float32),
            pltpu.VMEM((QTR, D), jnp.float32),
            pltpu.VMEM((QTR, D), jnp.float32),
            pltpu.VMEM((QTR, D), jnp.float32),
            pltpu.VMEM((QTR, D), jnp.float32),
            pltpu.VMEM((QTR, D), jnp.float32),
            [pltpu.SemaphoreType.DMA] * 4,
        ],
    )(contrib, d0, d1)


@jax.jit
def kernel(x, router_logits_up, router_logits_gate, router_logits_down, w_up, w_gate, w_down):
    del router_logits_gate, router_logits_down
    d0, d1, w0, w1, be = _dispatch(router_logits_up.T)
    xs, ws = _scatter(x, d0, d1, w0, w1)
    contrib = _gemm(be, xs, ws, w_up, w_gate, w_down)
    return _combine(contrib, d0, d1)


# speedup vs baseline: 1.6953x; 1.0345x over previous
import jax
import jax.numpy as jnp
from jax import lax
from jax.experimental import pallas as pl
from jax.experimental.pallas import tpu as pltpu
from jax.experimental.pallas import tpu_sc as plsc

E = 8
D = 1024
FF = 512
T = 2048
BLOCK = 256
NBMAX = (2 * T) // BLOCK + E - 1
P = NBMAX * BLOCK
NC = 2
NS = 16
NW = NC * NS
TPW = T // NW
BE_LEN = 48


def _dispatch_body(logits_ref, d0_ref, d1_ref, w0_ref, w1_ref, be_ref):
    logits = logits_ref[...].astype(jnp.float32)
    sub = lax.broadcasted_iota(jnp.int32, (E, T), 0)
    probs = jax.nn.softmax(logits, axis=0)
    m1 = jnp.max(probs, axis=0, keepdims=True)
    i1 = jnp.min(jnp.where(probs == m1, sub, E), axis=0, keepdims=True)
    probs2 = jnp.where(sub == i1, -1.0, probs)
    m2 = jnp.max(probs2, axis=0, keepdims=True)
    i2 = jnp.min(jnp.where(probs2 == m2, sub, E), axis=0, keepdims=True)
    s = m1 + m2
    w0_ref[...] = (m1 / s).reshape(T)
    w1_ref[...] = (m2 / s).reshape(T)

    mmat = ((sub == i1) | (sub == i2)).astype(jnp.int32)
    counts = jnp.sum(mmat, axis=1, keepdims=True)
    nb = (counts + (BLOCK - 1)) >> 8

    cum = nb
    for sh in (1, 2, 4):
        cum = cum + jnp.concatenate(
            [jnp.zeros((sh, 1), jnp.int32), cum[: E - sh, :]], axis=0)
    starts = cum - nb
    row_starts = starts * BLOCK

    inc = mmat
    sh = 1
    while sh < T:
        inc = inc + jnp.concatenate(
            [jnp.zeros((E, sh), jnp.int32), inc[:, : T - sh]], axis=1)
        sh *= 2
    prior = inc - mmat
    slot = prior + row_starts
    d0_ref[...] = jnp.sum(jnp.where(sub == i1, slot, 0), axis=0).reshape(T)
    d1_ref[...] = jnp.sum(jnp.where(sub == i2, slot, 0), axis=0).reshape(T)

    sub8 = lax.broadcasted_iota(jnp.int32, (E, 1), 0)
    b_iota = lax.broadcasted_iota(jnp.int32, (1, BE_LEN), 1)
    cnt = jnp.zeros((1, BE_LEN), jnp.int32)
    for e in range(E):
        s_e = jnp.sum(jnp.where(sub8 == e, starts, 0))
        cnt = cnt + (s_e <= b_iota).astype(jnp.int32)
    nbt = jnp.sum(nb)
    be_ref[...] = jnp.where(b_iota == BE_LEN - 1, nbt, cnt - 1).reshape(BE_LEN)


def _dispatch(logits_t):
    return pl.pallas_call(
        _dispatch_body,
        out_shape=(
            jax.ShapeDtypeStruct((T,), jnp.int32),
            jax.ShapeDtypeStruct((T,), jnp.int32),
            jax.ShapeDtypeStruct((T,), jnp.float32),
            jax.ShapeDtypeStruct((T,), jnp.float32),
            jax.ShapeDtypeStruct((BE_LEN,), jnp.int32),
        ),
    )(logits_t)


HPW = TPW // 2


def _scatter_body(x_hbm, d0_hbm, d1_hbm, w0_hbm, w1_hbm, xs_hbm, ws_hbm,
                  xr0, xr1, d0a, d0b, d1a, d1b, w0a, w0b, w1a, w1b, sem, wsem):
    wid = lax.axis_index("c") * NS + lax.axis_index("s")
    base = wid * TPW
    lx0 = pltpu.async_copy(x_hbm.at[pl.ds(base, HPW)], xr0, sem)
    lx1 = pltpu.async_copy(x_hbm.at[pl.ds(base + HPW, HPW)], xr1, sem)
    smalls = [
        pltpu.async_copy(d0_hbm.at[pl.ds(base, HPW)], d0a, wsem),
        pltpu.async_copy(d0_hbm.at[pl.ds(base + HPW, HPW)], d0b, wsem),
        pltpu.async_copy(d1_hbm.at[pl.ds(base, HPW)], d1a, wsem),
        pltpu.async_copy(d1_hbm.at[pl.ds(base + HPW, HPW)], d1b, wsem),
        pltpu.async_copy(w0_hbm.at[pl.ds(base, HPW)], w0a, wsem),
        pltpu.async_copy(w0_hbm.at[pl.ds(base + HPW, HPW)], w0b, wsem),
        pltpu.async_copy(w1_hbm.at[pl.ds(base, HPW)], w1a, wsem),
        pltpu.async_copy(w1_hbm.at[pl.ds(base + HPW, HPW)], w1b, wsem),
    ]
    for cp in smalls:
        cp.wait()
    stores = [
        pltpu.async_copy(w0a, ws_hbm.at[d0a], wsem),
        pltpu.async_copy(w0b, ws_hbm.at[d0b], wsem),
        pltpu.async_copy(w1a, ws_hbm.at[d1a], wsem),
        pltpu.async_copy(w1b, ws_hbm.at[d1b], wsem),
    ]
    lx0.wait()
    stores += [
        pltpu.async_copy(xr0, xs_hbm.at[d0a], sem),
        pltpu.async_copy(xr0, xs_hbm.at[d1a], sem),
    ]
    lx1.wait()
    stores += [
        pltpu.async_copy(xr1, xs_hbm.at[d0b], sem),
        pltpu.async_copy(xr1, xs_hbm.at[d1b], sem),
    ]
    for cp in stores:
        cp.wait()


def _scatter(x, d0, d1, w0, w1):
    mesh = plsc.VectorSubcoreMesh(
        core_axis_name="c", subcore_axis_name="s", num_cores=NC, num_subcores=NS)
    return pl.kernel(
        _scatter_body,
        out_type=(
            jax.ShapeDtypeStruct((P, D), jnp.float32),
            jax.ShapeDtypeStruct((P,), jnp.float32),
        ),
        mesh=mesh,
        scratch_types=[
            pltpu.VMEM((HPW, D), jnp.float32),
            pltpu.VMEM((HPW, D), jnp.float32),
            pltpu.VMEM((HPW,), jnp.int32),
            pltpu.VMEM((HPW,), jnp.int32),
            pltpu.VMEM((HPW,), jnp.int32),
            pltpu.VMEM((HPW,), jnp.int32),
            pltpu.VMEM((HPW,), jnp.float32),
            pltpu.VMEM((HPW,), jnp.float32),
            pltpu.VMEM((HPW,), jnp.float32),
            pltpu.VMEM((HPW,), jnp.float32),
            pltpu.SemaphoreType.DMA,
            pltpu.SemaphoreType.DMA,
        ],
    )(x, d0, d1, w0, w1)


def _gemm_body(be_ref, xs_ref, ws_ref, wu_ref, wg_ref, wd_ref, out_ref):
    b = pl.program_id(0)

    @pl.when(b < be_ref[BE_LEN - 1])
    def _():
        w_row = ws_ref[0]
        subl = lax.broadcasted_iota(jnp.int32, (BLOCK, BLOCK), 0)
        lanes = lax.broadcasted_iota(jnp.int32, (BLOCK, BLOCK), 1)
        diag = jnp.where(subl == lanes,
                         jnp.broadcast_to(w_row, (BLOCK, BLOCK)), 0.0)
        w = jnp.sum(diag, axis=1, keepdims=True)
        xc = xs_ref[...].astype(jnp.bfloat16)
        wu = wu_ref[0].astype(jnp.bfloat16)
        wg = wg_ref[0].astype(jnp.bfloat16)
        wd = wd_ref[0].astype(jnp.bfloat16)
        dn = (((1,), (1,)), ((), ()))
        up = lax.dot_general(xc, wu, dn, preferred_element_type=jnp.float32)
        gate = lax.dot_general(xc, wg, dn, preferred_element_type=jnp.float32)
        gw = gate * w
        h = gw * jax.nn.sigmoid(gw) * (up * w)
        out_ref[...] = lax.dot_general(
            h.astype(jnp.bfloat16), wd, dn,
            preferred_element_type=jnp.float32) * w


def _gemm(be, xs, ws, w_up, w_gate, w_down):
    def clamp(b, be_ref):
        return jnp.minimum(b, be_ref[BE_LEN - 1] - 1)

    grid_spec = pltpu.PrefetchScalarGridSpec(
        num_scalar_prefetch=1,
        grid=(NBMAX,),
        in_specs=[
            pl.BlockSpec((BLOCK, D), lambda b, be: (clamp(b, be), 0)),
            pl.BlockSpec((1, 1, BLOCK), lambda b, be: (clamp(b, be), 0, 0)),
            pl.BlockSpec((1, FF, D), lambda b, be: (be[clamp(b, be)], 0, 0)),
            pl.BlockSpec((1, FF, D), lambda b, be: (be[clamp(b, be)], 0, 0)),
            pl.BlockSpec((1, D, FF), lambda b, be: (be[clamp(b, be)], 0, 0)),
        ],
        out_specs=pl.BlockSpec((BLOCK, D), lambda b, be: (clamp(b, be), 0)),
    )
    return pl.pallas_call(
        _gemm_body,
        grid_spec=grid_spec,
        out_shape=jax.ShapeDtypeStruct((P, D), jnp.float32),
    )(be, xs, ws.reshape(NBMAX, 1, BLOCK), w_up, w_gate, w_down)


QTR = TPW // 4


def _combine_body(contrib_hbm, d0_hbm, d1_hbm, out_hbm,
                  p0_v, p1_v, g0a, g1a, g0b, g1b, oa, ob, sems):
    wid = lax.axis_index("c") * NS + lax.axis_index("s")
    base = wid * TPW
    pltpu.sync_copy(d0_hbm.at[pl.ds(base, TPW)], p0_v)
    pltpu.sync_copy(d1_hbm.at[pl.ds(base, TPW)], p1_v)
    gsets = [(g0a, g1a, oa), (g0b, g1b, ob)]

    def issue(q, gset):
        g0, g1, _ = gset
        return (
            pltpu.async_copy(contrib_hbm.at[p0_v.at[pl.ds(q * QTR, QTR)]], g0, sems[0]),
            pltpu.async_copy(contrib_hbm.at[p1_v.at[pl.ds(q * QTR, QTR)]], g1, sems[1]),
        )

    pend = issue(0, gsets[0])
    outcp = [None, None]
    for q in range(4):
        s = q % 2
        g0, g1, ob_ = gsets[s]
        for cp in pend:
            cp.wait()
        if q + 1 < 4:
            pend = issue(q + 1, gsets[1 - s])
        if outcp[s] is not None:
            outcp[s].wait()

        def chunk(i, _):
            sl = pl.ds(i * 16, 16)
            for r in range(QTR):
                ob_[r, sl] = g0[r, sl] + g1[r, sl]
            return 0

        lax.fori_loop(0, D // 16, chunk, 0, unroll=False)
        outcp[s] = pltpu.async_copy(
            ob_, out_hbm.at[pl.ds(base + q * QTR, QTR)], sems[2 + s])
    for cp in outcp:
        cp.wait()


def _combine(contrib, d0, d1):
    mesh = plsc.VectorSubcoreMesh(
        core_axis_name="c", subcore_axis_name="s", num_cores=NC, num_subcores=NS)
    return pl.kernel(
        _combine_body,
        out_type=jax.ShapeDtypeStruct((T, D), jnp.float32),
        mesh=mesh,
        scratch_types=[
            pltpu.VMEM((TPW,), jnp.int32),
            pltpu.VMEM((TPW,), jnp.int32),
            pltpu.VMEM((QTR, D), jnp.float32),
            pltpu.VMEM((QTR, D), jnp.float32),
            pltpu.VMEM((QTR, D), jnp.float32),
            pltpu.VMEM((QTR, D), jnp.float32),
            pltpu.VMEM((QTR, D), jnp.float32),
            pltpu.VMEM((QTR, D), jnp.float32),
            [pltpu.SemaphoreType.DMA] * 4,
        ],
    )(contrib, d0, d1)


@jax.jit
def kernel(x, router_logits_up, router_logits_gate, router_logits_down, w_up, w_gate, w_down):
    del router_logits_gate, router_logits_down
    d0, d1, w0, w1, be = _dispatch(router_logits_up.T)
    xs, ws = _scatter(x, d0, d1, w0, w1)
    contrib = _gemm(be, xs, ws, w_up, w_gate, w_down)
    return _combine(contrib, d0, d1)
